# baseline (speedup 1.0000x reference)
import functools

import jax
import jax.numpy as jnp
from jax import lax
from jax.experimental import pallas as pl
from jax.experimental.pallas import tpu as pltpu
from jax.experimental.pallas import tpu_sc as plsc

_NC, _NS = 2, 16
_NW = _NC * _NS
_CH = 500
_NBUF = 4
_AHEAD = 2


def _make_seg_sum(n_nodes, n_edges, d):
    ept = n_edges // _NW
    nmain = ept // _CH
    assert ept * _NW == n_edges and nmain * _CH == ept
    assert nmain % _NBUF == 0
    mesh = plsc.VectorSubcoreMesh(core_axis_name="c", subcore_axis_name="s")

    @functools.partial(
        pl.kernel,
        out_type=jax.ShapeDtypeStruct((_NC, n_nodes, d), jnp.float32),
        mesh=mesh,
        scratch_types=[
            pltpu.VMEM_SHARED((n_nodes, d), jnp.float32),
            pltpu.VMEM((nmain, _CH), jnp.int32),
            pltpu.VMEM((nmain, _CH), jnp.int32),
            pltpu.VMEM((_NBUF, _CH, d), jnp.float32),
            pltpu.SemaphoreType.DMA,
            pltpu.SemaphoreType.DMA,
        ],
        compiler_params=pltpu.CompilerParams(use_tc_tiling_on_sc=False),
    )
    def seg(y_hbm, srcm_hbm, dstm_hbm, zero_hbm,
            out_hbm, acc, srcv, dstv, rows, gsem, ssem):
        c = lax.axis_index("c")
        s = lax.axis_index("s")
        wid = c * _NS + s

        @pl.when(s == 0)
        def _init():
            pltpu.sync_copy(zero_hbm, acc)

        pltpu.sync_copy(srcm_hbm.at[wid], srcv)
        pltpu.sync_copy(dstm_hbm.at[wid], dstv)

        for k0 in range(_AHEAD):
            pltpu.async_copy(y_hbm.at[srcv.at[k0]], rows.at[k0], gsem)
        plsc.subcore_barrier()

        dummy_g = y_hbm.at[pl.ds(0, _CH)]
        last = nmain - 1

        def step(k, p):
            rbuf = rows.at[p]
            pltpu.make_async_copy(dummy_g, rbuf, gsem).wait()
            pltpu.async_copy(rbuf, acc.at[dstv.at[k]], ssem, add=True)

            @pl.when(k >= 2)
            def _retire():
                pltpu.make_async_copy(dummy_g, rbuf, ssem).wait()

            @pl.when(k + _AHEAD <= last)
            def _prefetch():
                pltpu.async_copy(y_hbm.at[srcv.at[k + _AHEAD]],
                                 rows.at[(p + _AHEAD) % _NBUF], gsem)

        @pl.loop(0, nmain, step=_NBUF)
        def _grp(g):
            for p in range(_NBUF):
                step(g + p, p)

        for _ in range(2):
            pltpu.make_async_copy(dummy_g, rows.at[0], ssem).wait()

        plsc.subcore_barrier()

        @pl.when(s == 0)
        def _flush():
            pltpu.sync_copy(acc, out_hbm.at[c])

    return seg


_seg40 = _make_seg_sum(10000, 320000, 40)
_seg16 = _make_seg_sum(10000, 320000, 16)


def _mm1_body(x_ref, wl_ref, caug_ref, wr_ref, y_ref, r_ref):
    xb = x_ref[...]
    y_ref[...] = jnp.dot(xb, wl_ref[...],
                         preferred_element_type=jnp.float32) + caug_ref[...]
    r_ref[...] = jnp.dot(xb, wr_ref[...], preferred_element_type=jnp.float32)


def _mm1(x, wlt, caug, wrt):
    n, k = x.shape
    da = wlt.shape[1]
    return pl.pallas_call(
        _mm1_body,
        out_shape=[
            jax.ShapeDtypeStruct((n, da), jnp.float32),
            jax.ShapeDtypeStruct((n, da), jnp.float32),
        ],
    )(x, wlt, caug, wrt)


def _mid_body(m_ref, r1_ref, b1_ref, sel_ref, w2l_ref, w2r_ref,
              y2_ref, r2_ref, inv_ref):
    a = m_ref[0] + m_ref[1]
    cnt = jnp.dot(a, sel_ref[...],
                  preferred_element_type=jnp.float32)
    inv = 1.0 / jnp.maximum(cnt, 1.0)
    h = jnp.maximum(a * inv + b1_ref[...] + r1_ref[...], 0.0)
    y2_ref[...] = jnp.dot(h, w2l_ref[...], preferred_element_type=jnp.float32)
    r2_ref[...] = jnp.dot(h, w2r_ref[...], preferred_element_type=jnp.float32)
    inv_ref[...] = inv


def _mid(aggm, r1, b1, sel, w2lt, w2rt):
    n = r1.shape[0]
    o = w2lt.shape[1]
    return pl.pallas_call(
        _mid_body,
        out_shape=[
            jax.ShapeDtypeStruct((n, o), jnp.float32),
            jax.ShapeDtypeStruct((n, o), jnp.float32),
            jax.ShapeDtypeStruct((n, 1), jnp.float32),
        ],
    )(aggm, r1, b1, sel, w2lt, w2rt)


def _fin_body(a_ref, inv_ref, r2_ref, b2_ref, out_ref):
    out_ref[...] = ((a_ref[0] + a_ref[1]) * inv_ref[...]
                    + b2_ref[...] + r2_ref[...])


def _fin(agg2, inv, r2, b2):
    n, o = r2.shape
    return pl.pallas_call(
        _fin_body,
        out_shape=jax.ShapeDtypeStruct((n, o), jnp.float32),
    )(agg2, inv, r2, b2)


def kernel(x, edge_index, W1_l, b1_l, W1_r, W2_l, b2_l, W2_r):
    n, in_dim = x.shape
    h = W1_l.shape[0]
    o = W2_l.shape[0]
    e = edge_index.shape[1]
    ept = e // _NW
    nmain = ept // _CH
    da = h + 8

    er = edge_index.reshape(2, _NW, nmain, _CH)
    src_m = er[0]
    dst_m = er[1]

    pad8 = jnp.zeros((in_dim, 8), jnp.float32)
    wlt = jnp.concatenate([W1_l.T, pad8], axis=1)
    wrt = jnp.concatenate([W1_r.T, pad8], axis=1)
    caug = jnp.concatenate([jnp.zeros((1, h), jnp.float32),
                            jnp.ones((1, 8), jnp.float32)], axis=1)

    y1, r1 = _mm1(x, wlt, caug, wrt)

    z40 = jnp.zeros((n, da), jnp.float32)
    agg1 = _seg40(y1, src_m, dst_m, z40)

    sel = jnp.zeros((da, 1), jnp.float32).at[h, 0].set(1.0)
    b1p = jnp.concatenate([b1_l, jnp.zeros((8,), jnp.float32)]).reshape(1, da)
    pad8o = jnp.zeros((8, o), jnp.float32)
    w2lt = jnp.concatenate([W2_l.T, pad8o], axis=0)
    w2rt = jnp.concatenate([W2_r.T, pad8o], axis=0)

    y2, r2, inv = _mid(agg1, r1, b1p, sel, w2lt, w2rt)

    z16 = jnp.zeros((n, o), jnp.float32)
    agg2 = _seg16(y2, src_m, dst_m, z16)

    return _fin(agg2, inv, r2, b2_l.reshape(1, o))


# speedup vs baseline: 1.0285x; 1.0285x over previous
import functools

import jax
import jax.numpy as jnp
from jax import lax
from jax.experimental import pallas as pl
from jax.experimental.pallas import tpu as pltpu
from jax.experimental.pallas import tpu_sc as plsc

_NC, _NS = 2, 16
_NW = _NC * _NS
_CH = 128
_NBUF = 6
_AHEAD = 4


def _make_seg_sum(n_nodes, n_edges, d):
    ept = n_edges // _NW
    nmain = ept // _CH
    tail = ept - nmain * _CH
    assert ept * _NW == n_edges and nmain % _NBUF == 0 and tail % 8 == 0
    mesh = plsc.VectorSubcoreMesh(core_axis_name="c", subcore_axis_name="s")

    @functools.partial(
        pl.kernel,
        out_type=jax.ShapeDtypeStruct((_NC, n_nodes, d), jnp.float32),
        mesh=mesh,
        scratch_types=[
            pltpu.VMEM_SHARED((n_nodes, d), jnp.float32),
            pltpu.VMEM((nmain, _CH), jnp.int32),
            pltpu.VMEM((nmain, _CH), jnp.int32),
            pltpu.VMEM((tail,), jnp.int32),
            pltpu.VMEM((tail,), jnp.int32),
            pltpu.VMEM((_NBUF, _CH, d), jnp.float32),
            pltpu.VMEM((tail, d), jnp.float32),
            pltpu.SemaphoreType.DMA,
            pltpu.SemaphoreType.DMA,
        ],
        compiler_params=pltpu.CompilerParams(use_tc_tiling_on_sc=False),
    )
    def seg(y_hbm, srcm_hbm, dstm_hbm, srct_hbm, dstt_hbm, zero_hbm,
            out_hbm, acc, srcv, dstv, srct, dstt, rows, rowst, gsem, ssem):
        c = lax.axis_index("c")
        s = lax.axis_index("s")
        wid = c * _NS + s

        @pl.when(s == 0)
        def _init():
            pltpu.sync_copy(zero_hbm, acc)

        pltpu.sync_copy(srcm_hbm.at[wid], srcv)
        pltpu.sync_copy(dstm_hbm.at[wid], dstv)
        pltpu.sync_copy(srct_hbm.at[wid], srct)
        pltpu.sync_copy(dstt_hbm.at[wid], dstt)

        for k0 in range(_AHEAD):
            pltpu.async_copy(y_hbm.at[srcv.at[k0]], rows.at[k0], gsem)
        plsc.subcore_barrier()

        dummy_g = y_hbm.at[pl.ds(0, _CH)]
        last = nmain - 1

        def step(k, p):
            rbuf = rows.at[p]
            pltpu.make_async_copy(dummy_g, rbuf, gsem).wait()
            pltpu.async_copy(rbuf, acc.at[dstv.at[k]], ssem, add=True)

            @pl.when(k >= 2)
            def _retire():
                pltpu.make_async_copy(dummy_g, rbuf, ssem).wait()

            @pl.when(k + _AHEAD <= last)
            def _prefetch():
                pltpu.async_copy(y_hbm.at[srcv.at[k + _AHEAD]],
                                 rows.at[(p + _AHEAD) % _NBUF], gsem)

        @pl.loop(0, nmain, step=_NBUF)
        def _grp(g):
            for p in range(_NBUF):
                step(g + p, p)

        for _ in range(2):
            pltpu.make_async_copy(dummy_g, rows.at[0], ssem).wait()

        if tail:
            pltpu.async_copy(y_hbm.at[srct], rowst, gsem).wait()
            pltpu.sync_copy(rowst, acc.at[dstt], add=True)

        plsc.subcore_barrier()

        @pl.when(s == 0)
        def _flush():
            pltpu.sync_copy(acc, out_hbm.at[c])

    return seg


_seg40 = _make_seg_sum(10000, 320000, 40)
_seg16 = _make_seg_sum(10000, 320000, 16)


def _mm1_body(x_ref, wl_ref, caug_ref, wr_ref, y_ref, r_ref):
    xb = x_ref[...]
    y_ref[...] = jnp.dot(xb, wl_ref[...],
                         preferred_element_type=jnp.float32) + caug_ref[...]
    r_ref[...] = jnp.dot(xb, wr_ref[...], preferred_element_type=jnp.float32)


def _mm1(x, wlt, caug, wrt):
    n, k = x.shape
    da = wlt.shape[1]
    return pl.pallas_call(
        _mm1_body,
        out_shape=[
            jax.ShapeDtypeStruct((n, da), jnp.float32),
            jax.ShapeDtypeStruct((n, da), jnp.float32),
        ],
    )(x, wlt, caug, wrt)


def _mid_body(m_ref, r1_ref, b1_ref, sel_ref, w2l_ref, w2r_ref,
              y2_ref, r2_ref, inv_ref):
    a = m_ref[0] + m_ref[1]
    cnt = jnp.dot(a, sel_ref[...],
                  preferred_element_type=jnp.float32)
    inv = 1.0 / jnp.maximum(cnt, 1.0)
    h = jnp.maximum(a * inv + b1_ref[...] + r1_ref[...], 0.0)
    y2_ref[...] = jnp.dot(h, w2l_ref[...], preferred_element_type=jnp.float32)
    r2_ref[...] = jnp.dot(h, w2r_ref[...], preferred_element_type=jnp.float32)
    inv_ref[...] = inv


def _mid(aggm, r1, b1, sel, w2lt, w2rt):
    n = r1.shape[0]
    o = w2lt.shape[1]
    return pl.pallas_call(
        _mid_body,
        out_shape=[
            jax.ShapeDtypeStruct((n, o), jnp.float32),
            jax.ShapeDtypeStruct((n, o), jnp.float32),
            jax.ShapeDtypeStruct((n, 1), jnp.float32),
        ],
    )(aggm, r1, b1, sel, w2lt, w2rt)


def _fin_body(a_ref, inv_ref, r2_ref, b2_ref, out_ref):
    out_ref[...] = ((a_ref[0] + a_ref[1]) * inv_ref[...]
                    + b2_ref[...] + r2_ref[...])


def _fin(agg2, inv, r2, b2):
    n, o = r2.shape
    return pl.pallas_call(
        _fin_body,
        out_shape=jax.ShapeDtypeStruct((n, o), jnp.float32),
    )(agg2, inv, r2, b2)


def kernel(x, edge_index, W1_l, b1_l, W1_r, W2_l, b2_l, W2_r):
    n, in_dim = x.shape
    h = W1_l.shape[0]
    o = W2_l.shape[0]
    e = edge_index.shape[1]
    ept = e // _NW
    nmain = ept // _CH
    da = h + 8

    er = edge_index.reshape(2, _NW, ept)
    src_m = er[0, :, :nmain * _CH].reshape(_NW, nmain, _CH)
    dst_m = er[1, :, :nmain * _CH].reshape(_NW, nmain, _CH)
    src_t = er[0, :, nmain * _CH:]
    dst_t = er[1, :, nmain * _CH:]

    pad8 = jnp.zeros((in_dim, 8), jnp.float32)
    wlt = jnp.concatenate([W1_l.T, pad8], axis=1)
    wrt = jnp.concatenate([W1_r.T, pad8], axis=1)
    caug = jnp.concatenate([jnp.zeros((1, h), jnp.float32),
                            jnp.ones((1, 8), jnp.float32)], axis=1)

    y1, r1 = _mm1(x, wlt, caug, wrt)

    z40 = jnp.zeros((n, da), jnp.float32)
    agg1 = _seg40(y1, src_m, dst_m, src_t, dst_t, z40)

    sel = jnp.zeros((da, 1), jnp.float32).at[h, 0].set(1.0)
    b1p = jnp.concatenate([b1_l, jnp.zeros((8,), jnp.float32)]).reshape(1, da)
    pad8o = jnp.zeros((8, o), jnp.float32)
    w2lt = jnp.concatenate([W2_l.T, pad8o], axis=0)
    w2rt = jnp.concatenate([W2_r.T, pad8o], axis=0)

    y2, r2, inv = _mid(agg1, r1, b1p, sel, w2lt, w2rt)

    z16 = jnp.zeros((n, o), jnp.float32)
    agg2 = _seg16(y2, src_m, dst_m, src_t, dst_t, z16)

    return _fin(agg2, inv, r2, b2_l.reshape(1, o))
